# baseline (speedup 1.0000x reference)
import functools

import jax
import jax.numpy as jnp
from jax import lax
from jax.experimental import pallas as pl
from jax.experimental.pallas import tpu as pltpu
from jax.experimental.pallas import tpu_sc as plsc

N_NODES = 10000
N_EDGES = 160000
D_MODEL = 256
N_HEADS = 8
D_HEAD = D_MODEL // N_HEADS
INV_SQRT_DH = 1.0 / (D_HEAD ** 0.5)

NC, NS, LANES = 2, 16, 16
HALF = D_MODEL // NC
HEADS_PER_SC = N_HEADS // NC
ACCW = HALF + LANES
CH = 32
NCHUNK = N_EDGES // CH
NPT = ((NCHUNK + NS - 1) // NS + 3) // 4 * 4
ZROWS = 40
NZCH = N_NODES // ZROWS
ZCH_PER_TILE = -(-NZCH // NS)

BM = 400
NBLK = N_NODES // BM


def _qkv_body(x_ref, wq_ref, wk_ref, wv_ref, bq_ref, bk_ref, bv_ref,
              q_ref, k_ref, v_ref):
    xb = x_ref[...]
    q_ref[...] = jnp.dot(xb, wq_ref[...],
                         preferred_element_type=jnp.float32) + bq_ref[...]
    k_ref[...] = jnp.dot(xb, wk_ref[...],
                         preferred_element_type=jnp.float32) + bk_ref[...]
    v_ref[...] = jnp.dot(xb, wv_ref[...],
                         preferred_element_type=jnp.float32) + bv_ref[...]


def _qkv(x, Wq, Wk, Wv, bq, bk, bv):
    out_sh = jax.ShapeDtypeStruct((NC * N_NODES, HALF), jnp.float32)
    return pl.pallas_call(
        _qkv_body,
        grid=(NC, NBLK),
        in_specs=[
            pl.BlockSpec((BM, D_MODEL), lambda c, i: (i, 0)),
            pl.BlockSpec((D_MODEL, HALF), lambda c, i: (0, c)),
            pl.BlockSpec((D_MODEL, HALF), lambda c, i: (0, c)),
            pl.BlockSpec((D_MODEL, HALF), lambda c, i: (0, c)),
            pl.BlockSpec((1, HALF), lambda c, i: (0, c)),
            pl.BlockSpec((1, HALF), lambda c, i: (0, c)),
            pl.BlockSpec((1, HALF), lambda c, i: (0, c)),
        ],
        out_specs=[
            pl.BlockSpec((BM, HALF), lambda c, i: (c * NBLK + i, 0)),
            pl.BlockSpec((BM, HALF), lambda c, i: (c * NBLK + i, 0)),
            pl.BlockSpec((BM, HALF), lambda c, i: (c * NBLK + i, 0)),
        ],
        out_shape=[out_sh, out_sh, out_sh],
    )(x, Wq, Wk, Wv, bq.reshape(1, D_MODEL), bk.reshape(1, D_MODEL),
      bv.reshape(1, D_MODEL))


def _edge_body(qh, kh, vh, src, dst, out,
               srcv0, dstv0, srcov0, dstov0, dsc0, qv0, kv0, vv0, msgv0,
               srcv1, dstv1, srcov1, dstov1, dsc1, qv1, kv1, vv1, msgv1,
               zb, acc_sh,
               semg0, semi0, sems0, semg1, semi1, sems1):
    c = lax.axis_index("c")
    s = lax.axis_index("s")
    c_n = c * N_NODES
    zero16 = jnp.zeros((LANES,), jnp.float32)
    lane = lax.iota(jnp.int32, LANES)
    bufs = (
        (srcv0, dstv0, srcov0, dstov0, dsc0, qv0, kv0, vv0, msgv0,
         semg0, semi0, sems0),
        (srcv1, dstv1, srcov1, dstov1, dsc1, qv1, kv1, vv1, msgv1,
         semg1, semi1, sems1),
    )

    def _zrow(r, _):
        for t in range(ACCW // LANES):
            zb[r, pl.ds(t * LANES, LANES)] = zero16
        return 0
    lax.fori_loop(0, ZROWS, _zrow, 0, unroll=False)

    def _zcopy(b, _):
        ch = s + NS * b

        @pl.when(ch < NZCH)
        def _():
            pltpu.sync_copy(zb, acc_sh.at[pl.ds(ch * ZROWS, ZROWS)])

        return 0
    lax.fori_loop(0, ZCH_PER_TILE, _zcopy, 0, unroll=False)
    plsc.subcore_barrier()

    def _base(jj):
        return jnp.minimum(s + NS * jj, NCHUNK - 1) * CH

    def _offsets(bset):
        srcv, dstv, srcov, dstov, dsc = bset[:5]
        for t in range(CH // LANES):
            sl = pl.ds(t * LANES, LANES)
            sv = srcv[sl]
            dv = dstv[sl]
            srcov[sl] = sv + c_n
            dstov[sl] = dv + c_n
            dsc[sl] = dv

    def _issue_gathers(bset):
        _, _, srcov, dstov, _, qv, kv, vv, _, semg, _, _ = bset
        return (pltpu.async_copy(qh.at[dstov], qv, semg),
                pltpu.async_copy(kh.at[srcov], kv, semg),
                pltpu.async_copy(vh.at[srcov], vv, semg))

    def _issue_idx(bset, base):
        srcv, dstv, semi = bset[0], bset[1], bset[10]
        pltpu.async_copy(src.at[pl.ds(base, CH)], srcv, semi)
        pltpu.async_copy(dst.at[pl.ds(base, CH)], dstv, semi)

    def _wait_idx(bset):
        srcv, dstv, semi = bset[0], bset[1], bset[10]
        pltpu.make_async_copy(src.at[pl.ds(0, CH)], srcv, semi).wait()
        pltpu.make_async_copy(dst.at[pl.ds(0, CH)], dstv, semi).wait()

    def _issue_scatter(bset):
        dsc, msgv, sems = bset[4], bset[8], bset[11]
        return pltpu.async_copy(msgv, acc_sh.at[dsc], sems, add=True)

    def _compute_msgs(bset, jj):
        qv, kv, vv, msgv = bset[5], bset[6], bset[7], bset[8]
        g = s + NS * jj
        scale = jnp.where(g < NCHUNK, 1.0, 0.0)
        scale_v = jnp.full((LANES,), scale, jnp.float32)

        @plsc.parallel_loop(0, CH, unroll=4)
        def _edge(e):
            prods = []
            for r in range(HALF // LANES):
                sl = pl.ds(r * LANES, LANES)
                prods.append(qv[e, sl] * kv[e, sl])
            evecs = []
            for h in range(HEADS_PER_SC):
                t = prods[2 * h] + prods[2 * h + 1]
                sc = jnp.sum(t) * INV_SQRT_DH
                ev = jnp.exp(jnp.full((LANES,), sc, jnp.float32)) * scale_v
                lo = pl.ds(D_HEAD * h, LANES)
                hi = pl.ds(D_HEAD * h + LANES, LANES)
                msgv[e, lo] = ev * vv[e, lo]
                msgv[e, hi] = ev * vv[e, hi]
                evecs.append(ev)
            dvec = jnp.where(
                lane == 0, evecs[0],
                jnp.where(lane == 1, evecs[1],
                          jnp.where(lane == 2, evecs[2],
                                    jnp.where(lane == 3, evecs[3],
                                              zero16))))
            msgv[e, pl.ds(HALF, LANES)] = dvec

    A, B = bufs
    _issue_idx(A, _base(0))
    _issue_idx(B, _base(1))

    def _quad(j, _):
        ja = 4 * j
        _wait_idx(A)
        _offsets(A)
        cp0 = _issue_gathers(A)
        _issue_idx(A, _base(ja + 2))
        _wait_idx(B)
        _offsets(B)
        cp1 = _issue_gathers(B)
        _issue_idx(B, _base(ja + 3))
        for d in cp0:
            d.wait()
        _compute_msgs(A, ja)
        s0 = _issue_scatter(A)
        for d in cp1:
            d.wait()
        _compute_msgs(B, ja + 1)
        s1 = _issue_scatter(B)
        s0.wait()
        _wait_idx(A)
        _offsets(A)
        cp2 = _issue_gathers(A)
        _issue_idx(A, _base(ja + 4))
        s1.wait()
        _wait_idx(B)
        _offsets(B)
        cp3 = _issue_gathers(B)
        _issue_idx(B, _base(ja + 5))
        for d in cp2:
            d.wait()
        _compute_msgs(A, ja + 2)
        s2 = _issue_scatter(A)
        for d in cp3:
            d.wait()
        _compute_msgs(B, ja + 3)
        s3 = _issue_scatter(B)
        s2.wait()
        s3.wait()
        return 0

    lax.fori_loop(0, NPT // 4, _quad, 0, unroll=False)

    _wait_idx(A)
    _wait_idx(B)
    plsc.subcore_barrier()

    def _ocopy(b, _):
        ch = s + NS * b

        @pl.when(ch < NZCH)
        def _():
            row0 = ch * ZROWS
            pltpu.sync_copy(acc_sh.at[pl.ds(row0, ZROWS)], zb)
            pltpu.sync_copy(zb, out.at[pl.ds(c_n + row0, ZROWS)])

        return 0
    lax.fori_loop(0, ZCH_PER_TILE, _ocopy, 0, unroll=False)


def _edge_pass(qh, kh, vh, src, dst):
    mesh = plsc.VectorSubcoreMesh(core_axis_name="c", subcore_axis_name="s")
    idx_t = pltpu.VMEM((CH,), jnp.int32)
    row_t = pltpu.VMEM((CH, HALF), jnp.float32)
    msg_t = pltpu.VMEM((CH, ACCW), jnp.float32)
    fn = pl.kernel(
        _edge_body,
        out_type=jax.ShapeDtypeStruct((NC * N_NODES, ACCW), jnp.float32),
        mesh=mesh,
        scratch_types=(
            [idx_t] * 5 + [row_t] * 3 + [msg_t]
            + [idx_t] * 5 + [row_t] * 3 + [msg_t]
            + [pltpu.VMEM((ZROWS, ACCW), jnp.float32),
               pltpu.VMEM_SHARED((N_NODES, ACCW), jnp.float32)]
            + [pltpu.SemaphoreType.DMA] * 6
        ),
        compiler_params=pltpu.CompilerParams(
            needs_layout_passes=False, use_tc_tiling_on_sc=False),
    )
    return fn(qh, kh, vh, src, dst)


def _fin_body(a0_ref, a1_ref, x_ref, wo_ref, bo_ref, skip_ref,
              lng_ref, lnb_ref, o_ref):
    a0 = a0_ref[...]
    a1 = a1_ref[...]
    parts = []
    for a in (a0, a1):
        for h in range(HEADS_PER_SC):
            num = a[:, D_HEAD * h:D_HEAD * (h + 1)]
            den = a[:, HALF + h:HALF + h + 1] + 1e-16
            parts.append(num / den)
    agg = jnp.concatenate(parts, axis=1)
    hval = jnp.dot(agg, wo_ref[...],
                   preferred_element_type=jnp.float32) + bo_ref[...]
    g = jax.nn.sigmoid(skip_ref[0, 0])
    out = g * hval + (1.0 - g) * x_ref[...]
    mean = jnp.mean(out, axis=1, keepdims=True)
    ctr = out - mean
    var = jnp.mean(ctr * ctr, axis=1, keepdims=True)
    o_ref[...] = ctr * jax.lax.rsqrt(var + 1e-5) * lng_ref[...] + lnb_ref[...]


def _finalize(acc, x, Wo, bo, skip, ln_g, ln_b):
    return pl.pallas_call(
        _fin_body,
        grid=(NBLK,),
        in_specs=[
            pl.BlockSpec((BM, ACCW), lambda i: (i, 0)),
            pl.BlockSpec((BM, ACCW), lambda i: (NBLK + i, 0)),
            pl.BlockSpec((BM, D_MODEL), lambda i: (i, 0)),
            pl.BlockSpec((D_MODEL, D_MODEL), lambda i: (0, 0)),
            pl.BlockSpec((1, D_MODEL), lambda i: (0, 0)),
            pl.BlockSpec((1, 1), lambda i: (0, 0)),
            pl.BlockSpec((1, D_MODEL), lambda i: (0, 0)),
            pl.BlockSpec((1, D_MODEL), lambda i: (0, 0)),
        ],
        out_specs=pl.BlockSpec((BM, D_MODEL), lambda i: (i, 0)),
        out_shape=jax.ShapeDtypeStruct((N_NODES, D_MODEL), jnp.float32),
    )(acc, acc, x, Wo, bo.reshape(1, D_MODEL), skip.reshape(1, 1),
      ln_g.reshape(1, D_MODEL), ln_b.reshape(1, D_MODEL))


def kernel(x, edge_index, Wq, bq, Wk, bk, Wv, bv, Wo, bo, skip, ln_g, ln_b):
    src = edge_index[0]
    dst = edge_index[1]
    qh, kh, vh = _qkv(x, Wq, Wk, Wv, bq, bk, bv)
    acc = _edge_pass(qh, kh, vh, src, dst)
    return _finalize(acc, x, Wo, bo, skip, ln_g, ln_b)


# speedup vs baseline: 1.1892x; 1.1892x over previous
import functools

import jax
import jax.numpy as jnp
from jax import lax
from jax.experimental import pallas as pl
from jax.experimental.pallas import tpu as pltpu
from jax.experimental.pallas import tpu_sc as plsc

N_NODES = 10000
N_EDGES = 160000
D_MODEL = 256
N_HEADS = 8
D_HEAD = D_MODEL // N_HEADS
INV_SQRT_DH = 1.0 / (D_HEAD ** 0.5)

NC, NS, LANES = 2, 16, 16
HALF = D_MODEL // NC
HEADS_PER_SC = N_HEADS // NC
ACCW = HALF + LANES
CH = 32
NCHUNK = N_EDGES // CH
NPT = ((NCHUNK + NS - 1) // NS + 1) // 2 * 2
ZROWS = 40
NZCH = N_NODES // ZROWS
ZCH_PER_TILE = -(-NZCH // NS)

BM = 400
NBLK = N_NODES // BM


def _qkv_body(x_ref, wq_ref, wk_ref, wv_ref, bq_ref, bk_ref, bv_ref,
              q_ref, k_ref, v_ref):
    xb = x_ref[...]
    q_ref[...] = jnp.dot(xb, wq_ref[...],
                         preferred_element_type=jnp.float32) + bq_ref[...]
    k_ref[...] = jnp.dot(xb, wk_ref[...],
                         preferred_element_type=jnp.float32) + bk_ref[...]
    v_ref[...] = jnp.dot(xb, wv_ref[...],
                         preferred_element_type=jnp.float32) + bv_ref[...]


def _qkv(x, Wq, Wk, Wv, bq, bk, bv):
    out_sh = jax.ShapeDtypeStruct((NC * N_NODES, HALF), jnp.float32)
    return pl.pallas_call(
        _qkv_body,
        grid=(NC, NBLK),
        in_specs=[
            pl.BlockSpec((BM, D_MODEL), lambda c, i: (i, 0)),
            pl.BlockSpec((D_MODEL, HALF), lambda c, i: (0, c)),
            pl.BlockSpec((D_MODEL, HALF), lambda c, i: (0, c)),
            pl.BlockSpec((D_MODEL, HALF), lambda c, i: (0, c)),
            pl.BlockSpec((1, HALF), lambda c, i: (0, c)),
            pl.BlockSpec((1, HALF), lambda c, i: (0, c)),
            pl.BlockSpec((1, HALF), lambda c, i: (0, c)),
        ],
        out_specs=[
            pl.BlockSpec((BM, HALF), lambda c, i: (c * NBLK + i, 0)),
            pl.BlockSpec((BM, HALF), lambda c, i: (c * NBLK + i, 0)),
            pl.BlockSpec((BM, HALF), lambda c, i: (c * NBLK + i, 0)),
        ],
        out_shape=[out_sh, out_sh, out_sh],
    )(x, Wq, Wk, Wv, bq.reshape(1, D_MODEL), bk.reshape(1, D_MODEL),
      bv.reshape(1, D_MODEL))


def _edge_body(qh, kh, vh, src, dst, out,
               srcv0, dstv0, srcov0, dstov0, dsc0, qv0, kv0, vv0, msgv0,
               srcv1, dstv1, srcov1, dstov1, dsc1, qv1, kv1, vv1, msgv1,
               zb, acc_sh,
               semg0, semi0, sems0, semg1, semi1, sems1):
    c = lax.axis_index("c")
    s = lax.axis_index("s")
    c_n = c * N_NODES
    zero16 = jnp.zeros((LANES,), jnp.float32)
    lane = lax.iota(jnp.int32, LANES)
    bufs = (
        (srcv0, dstv0, srcov0, dstov0, dsc0, qv0, kv0, vv0, msgv0,
         semg0, semi0, sems0),
        (srcv1, dstv1, srcov1, dstov1, dsc1, qv1, kv1, vv1, msgv1,
         semg1, semi1, sems1),
    )

    def _zrow(r, _):
        for t in range(ACCW // LANES):
            zb[r, pl.ds(t * LANES, LANES)] = zero16
        return 0
    lax.fori_loop(0, ZROWS, _zrow, 0, unroll=False)

    def _zcopy(b, _):
        ch = s + NS * b

        @pl.when(ch < NZCH)
        def _():
            pltpu.sync_copy(zb, acc_sh.at[pl.ds(ch * ZROWS, ZROWS)])

        return 0
    lax.fori_loop(0, ZCH_PER_TILE, _zcopy, 0, unroll=False)
    plsc.subcore_barrier()

    def _base(jj):
        return jnp.minimum(s + NS * jj, NCHUNK - 1) * CH

    def _offsets(bset):
        srcv, dstv, srcov, dstov, dsc = bset[:5]
        for t in range(CH // LANES):
            sl = pl.ds(t * LANES, LANES)
            sv = srcv[sl]
            dv = dstv[sl]
            srcov[sl] = sv + c_n
            dstov[sl] = dv + c_n
            dsc[sl] = dv

    def _issue_gathers(bset):
        _, _, srcov, dstov, _, qv, kv, vv, _, semg, _, _ = bset
        return (pltpu.async_copy(qh.at[dstov], qv, semg),
                pltpu.async_copy(kh.at[srcov], kv, semg),
                pltpu.async_copy(vh.at[srcov], vv, semg))

    def _issue_idx(bset, base):
        srcv, dstv, semi = bset[0], bset[1], bset[10]
        pltpu.async_copy(src.at[pl.ds(base, CH)], srcv, semi)
        pltpu.async_copy(dst.at[pl.ds(base, CH)], dstv, semi)

    def _wait_idx(bset):
        srcv, dstv, semi = bset[0], bset[1], bset[10]
        pltpu.make_async_copy(src.at[pl.ds(0, CH)], srcv, semi).wait()
        pltpu.make_async_copy(dst.at[pl.ds(0, CH)], dstv, semi).wait()

    def _issue_scatter(bset):
        dsc, msgv, sems = bset[4], bset[8], bset[11]
        return pltpu.async_copy(msgv, acc_sh.at[dsc], sems, add=True)

    def _compute_msgs(bset, jj):
        qv, kv, vv, msgv = bset[5], bset[6], bset[7], bset[8]
        g = s + NS * jj
        scale = jnp.where(g < NCHUNK, 1.0, 0.0)
        scale_v = jnp.full((LANES,), scale, jnp.float32)

        @plsc.parallel_loop(0, CH, unroll=4)
        def _edge(e):
            prods = []
            for r in range(HALF // LANES):
                sl = pl.ds(r * LANES, LANES)
                prods.append(qv[e, sl] * kv[e, sl])
            evecs = []
            for h in range(HEADS_PER_SC):
                t = prods[2 * h] + prods[2 * h + 1]
                sc = jnp.sum(t) * INV_SQRT_DH
                ev = jnp.exp(jnp.full((LANES,), sc, jnp.float32)) * scale_v
                lo = pl.ds(D_HEAD * h, LANES)
                hi = pl.ds(D_HEAD * h + LANES, LANES)
                msgv[e, lo] = ev * vv[e, lo]
                msgv[e, hi] = ev * vv[e, hi]
                evecs.append(ev)
            dvec = jnp.where(
                lane == 0, evecs[0],
                jnp.where(lane == 1, evecs[1],
                          jnp.where(lane == 2, evecs[2],
                                    jnp.where(lane == 3, evecs[3],
                                              zero16))))
            msgv[e, pl.ds(HALF, LANES)] = dvec

    A, B = bufs
    _issue_idx(A, _base(0))
    _issue_idx(B, _base(1))

    def _pair(j, _):
        ja = 2 * j
        _wait_idx(A)
        _offsets(A)
        cp_a = _issue_gathers(A)
        _wait_idx(B)
        _offsets(B)
        cp_b = _issue_gathers(B)
        _issue_idx(A, _base(ja + 2))
        _issue_idx(B, _base(ja + 3))
        for d in cp_a:
            d.wait()
        _compute_msgs(A, ja)
        sc_a = _issue_scatter(A)
        for d in cp_b:
            d.wait()
        _compute_msgs(B, ja + 1)
        sc_b = _issue_scatter(B)
        sc_a.wait()
        sc_b.wait()
        return 0

    lax.fori_loop(0, NPT // 2, _pair, 0, unroll=False)

    _wait_idx(A)
    _wait_idx(B)
    plsc.subcore_barrier()

    def _ocopy(b, _):
        ch = s + NS * b

        @pl.when(ch < NZCH)
        def _():
            row0 = ch * ZROWS
            pltpu.sync_copy(acc_sh.at[pl.ds(row0, ZROWS)], zb)
            pltpu.sync_copy(zb, out.at[pl.ds(c_n + row0, ZROWS)])

        return 0
    lax.fori_loop(0, ZCH_PER_TILE, _ocopy, 0, unroll=False)


def _edge_pass(qh, kh, vh, src, dst):
    mesh = plsc.VectorSubcoreMesh(core_axis_name="c", subcore_axis_name="s")
    idx_t = pltpu.VMEM((CH,), jnp.int32)
    row_t = pltpu.VMEM((CH, HALF), jnp.float32)
    msg_t = pltpu.VMEM((CH, ACCW), jnp.float32)
    fn = pl.kernel(
        _edge_body,
        out_type=jax.ShapeDtypeStruct((NC * N_NODES, ACCW), jnp.float32),
        mesh=mesh,
        scratch_types=(
            [idx_t] * 5 + [row_t] * 3 + [msg_t]
            + [idx_t] * 5 + [row_t] * 3 + [msg_t]
            + [pltpu.VMEM((ZROWS, ACCW), jnp.float32),
               pltpu.VMEM_SHARED((N_NODES, ACCW), jnp.float32)]
            + [pltpu.SemaphoreType.DMA] * 6
        ),
        compiler_params=pltpu.CompilerParams(
            needs_layout_passes=False, use_tc_tiling_on_sc=False),
    )
    return fn(qh, kh, vh, src, dst)


def _fin_body(a0_ref, a1_ref, x_ref, wo_ref, bo_ref, skip_ref,
              lng_ref, lnb_ref, o_ref):
    a0 = a0_ref[...]
    a1 = a1_ref[...]
    parts = []
    for a in (a0, a1):
        for h in range(HEADS_PER_SC):
            num = a[:, D_HEAD * h:D_HEAD * (h + 1)]
            den = a[:, HALF + h:HALF + h + 1] + 1e-16
            parts.append(num / den)
    agg = jnp.concatenate(parts, axis=1)
    hval = jnp.dot(agg, wo_ref[...],
                   preferred_element_type=jnp.float32) + bo_ref[...]
    g = jax.nn.sigmoid(skip_ref[0, 0])
    out = g * hval + (1.0 - g) * x_ref[...]
    mean = jnp.mean(out, axis=1, keepdims=True)
    ctr = out - mean
    var = jnp.mean(ctr * ctr, axis=1, keepdims=True)
    o_ref[...] = ctr * jax.lax.rsqrt(var + 1e-5) * lng_ref[...] + lnb_ref[...]


def _finalize(acc, x, Wo, bo, skip, ln_g, ln_b):
    return pl.pallas_call(
        _fin_body,
        grid=(NBLK,),
        in_specs=[
            pl.BlockSpec((BM, ACCW), lambda i: (i, 0)),
            pl.BlockSpec((BM, ACCW), lambda i: (NBLK + i, 0)),
            pl.BlockSpec((BM, D_MODEL), lambda i: (i, 0)),
            pl.BlockSpec((D_MODEL, D_MODEL), lambda i: (0, 0)),
            pl.BlockSpec((1, D_MODEL), lambda i: (0, 0)),
            pl.BlockSpec((1, 1), lambda i: (0, 0)),
            pl.BlockSpec((1, D_MODEL), lambda i: (0, 0)),
            pl.BlockSpec((1, D_MODEL), lambda i: (0, 0)),
        ],
        out_specs=pl.BlockSpec((BM, D_MODEL), lambda i: (i, 0)),
        out_shape=jax.ShapeDtypeStruct((N_NODES, D_MODEL), jnp.float32),
    )(acc, acc, x, Wo, bo.reshape(1, D_MODEL), skip.reshape(1, 1),
      ln_g.reshape(1, D_MODEL), ln_b.reshape(1, D_MODEL))


def kernel(x, edge_index, Wq, bq, Wk, bk, Wv, bv, Wo, bo, skip, ln_g, ln_b):
    src = edge_index[0]
    dst = edge_index[1]
    qh, kh, vh = _qkv(x, Wq, Wk, Wv, bq, bk, bv)
    acc = _edge_pass(qh, kh, vh, src, dst)
    return _finalize(acc, x, Wo, bo, skip, ln_g, ln_b)
